# baseline (speedup 1.0000x reference)
import functools

import jax
import jax.numpy as jnp
from jax import lax
from jax.experimental import pallas as pl
from jax.experimental.pallas import tpu as pltpu
from jax.experimental.pallas import tpu_sc as plsc

_N = 10000
_E = 320000
_D = 128
_R = 16

_NC = 2
_NS = 16
_NW = _NC * _NS
_EPW = _E // _NW
_C = 80
_NCHUNK = _EPW // _C
_NPAD = 10240
_RZ = _NPAD // _NS


def _mm_body(h_ref, w_ref, y_ref):
    y_ref[0] = jnp.dot(h_ref[...], w_ref[0], preferred_element_type=jnp.float32)


def _relation_matmul(node_features, weight):
    bn = 1000
    return pl.pallas_call(
        _mm_body,
        grid=(_R, _N // bn),
        in_specs=[
            pl.BlockSpec((bn, _D), lambda r, i: (i, 0)),
            pl.BlockSpec((1, _D, _D), lambda r, i: (r, 0, 0)),
        ],
        out_specs=pl.BlockSpec((1, bn, _D), lambda r, i: (r, i, 0)),
        out_shape=jax.ShapeDtypeStruct((_R, _N, _D), jnp.float32),
    )(node_features, weight)


def _sc_body(
    pack_hbm, normc_hbm, y_hbm, zeros_hbm, out_hbm,
    edat, norm, gidx, dgid, dsid, gbuf, sbuf,
    esem, gsem, ssem, acc,
):
    cid = lax.axis_index("c")
    sid = lax.axis_index("s")
    wid = sid * _NC + cid

    pltpu.sync_copy(zeros_hbm, acc.at[pl.ds(sid * _RZ, _RZ)])
    plsc.subcore_barrier()

    nblk = _E // _C

    def pref(p, k):
        blk = jnp.minimum(wid * _NCHUNK + k, nblk - 1)
        pltpu.async_copy(pack_hbm.at[blk], edat[p], esem[p])
        pltpu.async_copy(normc_hbm.at[blk], norm[p], esem[p])

    def sg(p, k):
        blk = wid * _NCHUNK + k
        pltpu.make_async_copy(pack_hbm.at[blk], edat[p], esem[p]).wait()
        pltpu.make_async_copy(normc_hbm.at[blk], norm[p], esem[p]).wait()
        for i in range(_C // 16):
            sl = pl.ds(i * 16, 16)
            gidx[p][sl] = edat[p][2, sl] * _N + edat[p][0, sl]
            dgid[p][sl] = edat[p][1, sl]
        pltpu.async_copy(y_hbm.at[gidx[p]], gbuf[p], gsem[p])

    def fin(p, wait_s):
        pltpu.make_async_copy(y_hbm.at[gidx[p]], gbuf[p], gsem[p]).wait()
        if wait_s:
            pltpu.make_async_copy(sbuf[p], acc.at[dsid[p]], ssem[p]).wait()

        def scale_body(e, c2):
            nv = plsc.load_gather(norm[p], [jnp.full((16,), e, jnp.int32)])
            for j in range(_D // 16):
                sl = pl.ds(j * 16, 16)
                sbuf[p][e, sl] = gbuf[p][e, sl] * nv
            return c2

        lax.fori_loop(0, _C, scale_body, 0)
        for i in range(_C // 16):
            sl = pl.ds(i * 16, 16)
            dsid[p][sl] = dgid[p][sl]
        pltpu.async_copy(sbuf[p], acc.at[dsid[p]], ssem[p], add=True)

    pref(0, 0)
    pref(1, 1)
    sg(0, 0)
    sg(1, 1)
    fin(0, False)
    pref(0, 2)
    sg(0, 2)
    fin(1, False)
    pref(1, 3)

    def pipe_body(i, carry):
        a = 2 * i
        b = a + 1
        sg(1, b)
        fin(0, True)
        pref(0, a + 2)
        sg(0, a + 2)
        fin(1, True)
        pref(1, b + 2)
        return carry

    lax.fori_loop(1, _NCHUNK // 2, pipe_body, 0)

    fin(0, True)
    pltpu.make_async_copy(pack_hbm.at[0], edat[1], esem[1]).wait()
    pltpu.make_async_copy(normc_hbm.at[0], norm[1], esem[1]).wait()
    pltpu.make_async_copy(sbuf[0], acc.at[dsid[0]], ssem[0]).wait()
    pltpu.make_async_copy(sbuf[1], acc.at[dsid[1]], ssem[1]).wait()

    plsc.subcore_barrier()

    last_full = _N // _RZ
    rem = _N - last_full * _RZ

    @pl.when(sid < last_full)
    def _():
        pltpu.sync_copy(
            acc.at[pl.ds(sid * _RZ, _RZ)], out_hbm.at[cid, pl.ds(sid * _RZ, _RZ)]
        )

    @pl.when(sid == last_full)
    def _():
        pltpu.sync_copy(
            acc.at[pl.ds(last_full * _RZ, rem)],
            out_hbm.at[cid, pl.ds(last_full * _RZ, rem)],
        )


@functools.cache
def _sc_scatter():
    return pl.kernel(
        _sc_body,
        out_type=jax.ShapeDtypeStruct((_NC, _N, _D), jnp.float32),
        mesh=plsc.VectorSubcoreMesh(
            core_axis_name="c", subcore_axis_name="s", num_cores=_NC, num_subcores=_NS
        ),
        compiler_params=pltpu.CompilerParams(needs_layout_passes=False),
        scratch_types=[
            (pltpu.VMEM((3, _C), jnp.int32),) * 2,
            (pltpu.VMEM((_C,), jnp.float32),) * 2,
            (pltpu.VMEM((_C,), jnp.int32),) * 2,
            (pltpu.VMEM((_C,), jnp.int32),) * 2,
            (pltpu.VMEM((_C,), jnp.int32),) * 2,
            (pltpu.VMEM((_C, _D), jnp.float32),) * 2,
            (pltpu.VMEM((_C, _D), jnp.float32),) * 2,
            (pltpu.SemaphoreType.DMA,) * 2,
            (pltpu.SemaphoreType.DMA,) * 2,
            (pltpu.SemaphoreType.DMA,) * 2,
            pltpu.VMEM_SHARED((_NPAD, _D), jnp.float32),
        ],
    )


def _add_body(p_ref, o_ref):
    o_ref[...] = p_ref[0] + p_ref[1]


def _merge_partials(partials):
    ba = 1000
    return pl.pallas_call(
        _add_body,
        grid=(_N // ba,),
        in_specs=[pl.BlockSpec((_NC, ba, _D), lambda i: (0, i, 0))],
        out_specs=pl.BlockSpec((ba, _D), lambda i: (i, 0)),
        out_shape=jax.ShapeDtypeStruct((_N, _D), jnp.float32),
    )(partials)


def kernel(node_features, edge_index, edge_type, edge_norm, weight):
    src = edge_index[0]
    dst = edge_index[1]
    pack = jnp.stack(
        [
            src.reshape(_E // _C, _C),
            dst.reshape(_E // _C, _C),
            edge_type.reshape(_E // _C, _C),
        ],
        axis=1,
    )
    normc = edge_norm.reshape(_E // _C, _C)
    y = _relation_matmul(node_features, weight).reshape(_R * _N, _D)
    zeros = jnp.zeros((_RZ, _D), jnp.float32)
    partials = _sc_scatter()(pack, normc, y, zeros)
    return _merge_partials(partials)


# speedup vs baseline: 1.0628x; 1.0628x over previous
import functools

import jax
import jax.numpy as jnp
from jax import lax
from jax.experimental import pallas as pl
from jax.experimental.pallas import tpu as pltpu
from jax.experimental.pallas import tpu_sc as plsc

_N = 10000
_E = 320000
_D = 128
_R = 16

_NC = 2
_NS = 16
_NW = _NC * _NS
_EPW = _E // _NW
_C = 80
_NCHUNK = _EPW // _C
_NPAD = 10240
_RZ = _NPAD // _NS


def _mm_body(h_ref, w_ref, y_ref):
    y_ref[0] = jnp.dot(h_ref[...], w_ref[0], preferred_element_type=jnp.float32)


def _relation_matmul(node_features, weight):
    bn = 1000
    return pl.pallas_call(
        _mm_body,
        grid=(_R, _N // bn),
        in_specs=[
            pl.BlockSpec((bn, _D), lambda r, i: (i, 0)),
            pl.BlockSpec((1, _D, _D), lambda r, i: (r, 0, 0)),
        ],
        out_specs=pl.BlockSpec((1, bn, _D), lambda r, i: (r, i, 0)),
        out_shape=jax.ShapeDtypeStruct((_R, _N, _D), jnp.float32),
    )(node_features, weight)


def _sc_body(pack_hbm, normc_hbm, y_hbm, zeros_hbm, out_hbm, edat_v, norm_v, g_v, d_v, rows_v, acc, sem):
    cid = lax.axis_index("c")
    sid = lax.axis_index("s")
    wid = sid * _NC + cid

    pltpu.sync_copy(zeros_hbm, acc.at[pl.ds(sid * _RZ, _RZ)])
    plsc.subcore_barrier()

    def chunk_body(k, carry):
        blk = wid * _NCHUNK + k
        pltpu.sync_copy(pack_hbm.at[blk], edat_v)
        pltpu.sync_copy(normc_hbm.at[blk], norm_v)
        for i in range(_C // 16):
            sl = pl.ds(i * 16, 16)
            g_v[sl] = edat_v[2, sl] * _N + edat_v[0, sl]
            d_v[sl] = edat_v[1, sl]
        pltpu.async_copy(y_hbm.at[g_v], rows_v, sem).wait()

        def scale_body(e, c2):
            nv = plsc.load_gather(norm_v, [jnp.full((16,), e, jnp.int32)])
            for j in range(_D // 16):
                sl = pl.ds(j * 16, 16)
                rows_v[e, sl] = rows_v[e, sl] * nv
            return c2

        lax.fori_loop(0, _C, scale_body, 0)
        pltpu.sync_copy(rows_v, acc.at[d_v], add=True)
        return carry

    lax.fori_loop(0, _NCHUNK, chunk_body, 0)

    plsc.subcore_barrier()

    last_full = _N // _RZ
    rem = _N - last_full * _RZ

    @pl.when(sid < last_full)
    def _():
        pltpu.sync_copy(
            acc.at[pl.ds(sid * _RZ, _RZ)], out_hbm.at[cid, pl.ds(sid * _RZ, _RZ)]
        )

    @pl.when(sid == last_full)
    def _():
        pltpu.sync_copy(
            acc.at[pl.ds(last_full * _RZ, rem)],
            out_hbm.at[cid, pl.ds(last_full * _RZ, rem)],
        )


@functools.cache
def _sc_scatter():
    return pl.kernel(
        _sc_body,
        out_type=jax.ShapeDtypeStruct((_NC, _N, _D), jnp.float32),
        mesh=plsc.VectorSubcoreMesh(
            core_axis_name="c", subcore_axis_name="s", num_cores=_NC, num_subcores=_NS
        ),
        compiler_params=pltpu.CompilerParams(needs_layout_passes=False),
        scratch_types=[
            pltpu.VMEM((3, _C), jnp.int32),
            pltpu.VMEM((_C,), jnp.float32),
            pltpu.VMEM((_C,), jnp.int32),
            pltpu.VMEM((_C,), jnp.int32),
            pltpu.VMEM((_C, _D), jnp.float32),
            pltpu.VMEM_SHARED((_NPAD, _D), jnp.float32),
            pltpu.SemaphoreType.DMA,
        ],
    )


def _add_body(p_ref, o_ref):
    o_ref[...] = p_ref[0] + p_ref[1]


def _merge_partials(partials):
    ba = 1000
    return pl.pallas_call(
        _add_body,
        grid=(_N // ba,),
        in_specs=[pl.BlockSpec((_NC, ba, _D), lambda i: (0, i, 0))],
        out_specs=pl.BlockSpec((ba, _D), lambda i: (i, 0)),
        out_shape=jax.ShapeDtypeStruct((_N, _D), jnp.float32),
    )(partials)


def kernel(node_features, edge_index, edge_type, edge_norm, weight):
    src = edge_index[0]
    dst = edge_index[1]
    pack = jnp.stack(
        [
            src.reshape(_E // _C, _C),
            dst.reshape(_E // _C, _C),
            edge_type.reshape(_E // _C, _C),
        ],
        axis=1,
    )
    normc = edge_norm.reshape(_E // _C, _C)
    y = _relation_matmul(node_features, weight).reshape(_R * _N, _D)
    zeros = jnp.zeros((_RZ, _D), jnp.float32)
    partials = _sc_scatter()(pack, normc, y, zeros)
    return _merge_partials(partials)


# speedup vs baseline: 1.4143x; 1.3307x over previous
import functools

import jax
import jax.numpy as jnp
from jax import lax
from jax.experimental import pallas as pl
from jax.experimental.pallas import tpu as pltpu
from jax.experimental.pallas import tpu_sc as plsc

_N = 10000
_E = 320000
_D = 128
_R = 16

_NC = 2
_NS = 16
_NW = _NC * _NS
_C = 80
_NCHUNK = _E // _C // _NW
_NPAD = 10240
_RZ = _NPAD // _NS


def _mm_body(h_ref, w_ref, y_ref):
    y_ref[0] = jnp.dot(h_ref[...], w_ref[0], preferred_element_type=jnp.float32)


def _relation_matmul(node_features, weight):
    bn = 1000
    return pl.pallas_call(
        _mm_body,
        grid=(_R, _N // bn),
        in_specs=[
            pl.BlockSpec((bn, _D), lambda r, i: (i, 0)),
            pl.BlockSpec((1, _D, _D), lambda r, i: (r, 0, 0)),
        ],
        out_specs=pl.BlockSpec((1, bn, _D), lambda r, i: (r, i, 0)),
        out_shape=jax.ShapeDtypeStruct((_R, _N, _D), jnp.float32),
    )(node_features, weight)


def _sc_body(pack_hbm, norm_hbm, y_hbm, zeros_hbm, out_hbm,
             pack_v, norm_v, rows_v, acc, gsem0, gsem1):
    cid = lax.axis_index("c")
    sid = lax.axis_index("s")
    wid = sid * _NC + cid
    base = wid * _NCHUNK
    sems = (gsem0, gsem1)

    pltpu.sync_copy(zeros_hbm, acc.at[pl.ds(sid * _RZ, _RZ)])
    plsc.subcore_barrier()

    def load(b, k):
        pltpu.sync_copy(pack_hbm.at[base + k], pack_v.at[b])
        pltpu.sync_copy(norm_hbm.at[base + k], norm_v.at[b])

    def start_gather(b):
        pltpu.async_copy(y_hbm.at[pack_v.at[b, 0]], rows_v.at[b], sems[b])

    def wait_gather(b):
        pltpu.make_async_copy(
            y_hbm.at[pl.ds(0, _C)], rows_v.at[b], sems[b]
        ).wait()

    def scale(b):
        def scale_body(i, carry):
            for u in range(4):
                e = i * 4 + u
                nv = plsc.load_gather(
                    norm_v,
                    [jnp.full((16,), b, jnp.int32), jnp.full((16,), e, jnp.int32)],
                )
                for j in range(_D // 16):
                    sl = pl.ds(j * 16, 16)
                    rows_v[b, e, sl] = rows_v[b, e, sl] * nv
            return carry

        lax.fori_loop(0, _C // 4, scale_body, 0)

    def scatter(b):
        pltpu.sync_copy(rows_v.at[b], acc.at[pack_v.at[b, 1]], add=True)

    for b in range(2):
        load(b, b)
        start_gather(b)

    def pair_body(i, carry):
        for b in range(2):
            k = i * 2 + b
            wait_gather(b)
            scale(b)
            scatter(b)

            @pl.when(k + 2 < _NCHUNK)
            def _():
                load(b, k + 2)
                start_gather(b)

        return carry

    lax.fori_loop(0, (_NCHUNK - 1) // 2, pair_body, 0)

    wait_gather(0)
    scale(0)
    scatter(0)

    plsc.subcore_barrier()

    last_full = _N // _RZ
    rem = _N - last_full * _RZ

    @pl.when(sid < last_full)
    def _():
        pltpu.sync_copy(
            acc.at[pl.ds(sid * _RZ, _RZ)], out_hbm.at[cid, pl.ds(sid * _RZ, _RZ)]
        )

    @pl.when(sid == last_full)
    def _():
        pltpu.sync_copy(
            acc.at[pl.ds(last_full * _RZ, rem)],
            out_hbm.at[cid, pl.ds(last_full * _RZ, rem)],
        )


@functools.cache
def _sc_scatter():
    return pl.kernel(
        _sc_body,
        out_type=jax.ShapeDtypeStruct((_NC, _N, _D), jnp.float32),
        mesh=plsc.VectorSubcoreMesh(
            core_axis_name="c", subcore_axis_name="s", num_cores=_NC, num_subcores=_NS
        ),
        compiler_params=pltpu.CompilerParams(needs_layout_passes=False),
        scratch_types=[
            pltpu.VMEM((2, 2, _C), jnp.int32),
            pltpu.VMEM((2, _C), jnp.float32),
            pltpu.VMEM((2, _C, _D), jnp.float32),
            pltpu.VMEM_SHARED((_NPAD, _D), jnp.float32),
            pltpu.SemaphoreType.DMA,
            pltpu.SemaphoreType.DMA,
        ],
    )


def _add_body(p_ref, o_ref):
    o_ref[...] = p_ref[0] + p_ref[1]


def _merge_partials(partials):
    ba = 1000
    return pl.pallas_call(
        _add_body,
        grid=(_N // ba,),
        in_specs=[pl.BlockSpec((_NC, ba, _D), lambda i: (0, i, 0))],
        out_specs=pl.BlockSpec((ba, _D), lambda i: (i, 0)),
        out_shape=jax.ShapeDtypeStruct((_N, _D), jnp.float32),
    )(partials)


def kernel(node_features, edge_index, edge_type, edge_norm, weight):
    src = edge_index[0]
    dst = edge_index[1]
    nchunks = _E // _C
    g = edge_type.astype(jnp.int32) * _N + src.astype(jnp.int32)
    pack = jnp.stack(
        [g.reshape(nchunks, _C), dst.astype(jnp.int32).reshape(nchunks, _C)],
        axis=1,
    )
    normc = edge_norm.reshape(nchunks, _C)
    y = _relation_matmul(node_features, weight).reshape(_R * _N, _D)
    zeros = jnp.zeros((_RZ, _D), jnp.float32)
    partials = _sc_scatter()(pack, normc, y, zeros)
    return _merge_partials(partials)


# speedup vs baseline: 1.5021x; 1.0621x over previous
import functools

import jax
import jax.numpy as jnp
from jax import lax
from jax.experimental import pallas as pl
from jax.experimental.pallas import tpu as pltpu
from jax.experimental.pallas import tpu_sc as plsc

_N = 10000
_E = 320000
_D = 128
_R = 16

_NC = 2
_NS = 16
_NW = _NC * _NS
_C = 112
_NCHUNK = 90
_EPW = _NCHUNK * _C
_EPAD = _EPW * _NW
_NR = 3
_NP = 6
_NPAD = 10112
_DUMP = _NPAD - 8
_RZ = _NPAD // _NS


def _mm_body(h_ref, w_ref, y_ref):
    y_ref[0] = jnp.dot(h_ref[...], w_ref[0], preferred_element_type=jnp.float32)


def _relation_matmul(node_features, weight):
    bn = 1000
    return pl.pallas_call(
        _mm_body,
        grid=(_R, _N // bn),
        in_specs=[
            pl.BlockSpec((bn, _D), lambda r, i: (i, 0)),
            pl.BlockSpec((1, _D, _D), lambda r, i: (r, 0, 0)),
        ],
        out_specs=pl.BlockSpec((1, bn, _D), lambda r, i: (r, i, 0)),
        out_shape=jax.ShapeDtypeStruct((_R, _N, _D), jnp.float32),
    )(node_features, weight)


def _sc_body(pack_hbm, norm_hbm, y_hbm, zeros_hbm, out_hbm,
             pack_v, norm_v, rows_v, acc, *sems):
    cid = lax.axis_index("c")
    sid = lax.axis_index("s")
    wid = sid * _NC + cid
    base = wid * _NCHUNK
    lsem = sems[0:_NP]
    gsem = sems[_NP:_NP + _NR]
    ssem = sems[_NP + _NR:_NP + 2 * _NR]

    pltpu.sync_copy(zeros_hbm, acc.at[pl.ds(sid * _RZ, _RZ)])
    plsc.subcore_barrier()

    def start_load(p, k):
        pltpu.async_copy(pack_hbm.at[base + k], pack_v.at[p], lsem[p])
        pltpu.async_copy(norm_hbm.at[base + k], norm_v.at[p], lsem[p])

    def wait_load(p):
        pltpu.make_async_copy(pack_hbm.at[0], pack_v.at[p], lsem[p]).wait()
        pltpu.make_async_copy(norm_hbm.at[0], norm_v.at[p], lsem[p]).wait()

    def start_gather(s, p):
        pltpu.async_copy(y_hbm.at[pack_v.at[p, 0]], rows_v.at[s], gsem[s])

    def wait_gather(s):
        pltpu.make_async_copy(y_hbm.at[pl.ds(0, _C)], rows_v.at[s], gsem[s]).wait()

    def start_scatter(s, p):
        pltpu.async_copy(rows_v.at[s], acc.at[pack_v.at[p, 1]], ssem[s], add=True)

    def wait_scatter(s):
        pltpu.make_async_copy(y_hbm.at[pl.ds(0, _C)], rows_v.at[s], ssem[s]).wait()

    def scale(s, p):
        def scale_body(i, carry):
            for u in range(4):
                e = i * 4 + u
                nv = plsc.load_gather(
                    norm_v,
                    [jnp.full((16,), p, jnp.int32), jnp.full((16,), e, jnp.int32)],
                )
                for j in range(_D // 16):
                    sl = pl.ds(j * 16, 16)
                    rows_v[s, e, sl] = rows_v[s, e, sl] * nv
            return carry

        lax.fori_loop(0, _C // 4, scale_body, 0)

    for k in range(4):
        start_load(k, k)
    wait_load(0)
    start_gather(0, 0)

    for k in range(6):
        s, p = k % _NR, k % _NP
        s1, p1 = (k + 1) % _NR, (k + 1) % _NP
        if k >= 2:
            wait_scatter(s1)
        wait_load(p1)
        start_gather(s1, p1)
        start_load((k + 4) % _NP, k + 4)
        wait_gather(s)
        scale(s, p)
        start_scatter(s, p)

    def six_body(q, carry):
        for u in range(6):
            k = q * 6 + u
            s, p = u % _NR, u
            s1, p1 = (u + 1) % _NR, (u + 1) % _NP

            @pl.when(k + 1 < _NCHUNK)
            def _():
                wait_scatter(s1)
                wait_load(p1)
                start_gather(s1, p1)

            @pl.when(k + 4 < _NCHUNK)
            def _():
                start_load((u + 4) % _NP, k + 4)

            wait_gather(s)
            scale(s, p)
            start_scatter(s, p)
        return carry

    lax.fori_loop(1, _NCHUNK // 6, six_body, 0)

    for s in range(_NR):
        wait_scatter(s)

    plsc.subcore_barrier()

    last_full = _N // _RZ
    rem = _N - last_full * _RZ

    @pl.when(sid < last_full)
    def _():
        pltpu.sync_copy(
            acc.at[pl.ds(sid * _RZ, _RZ)], out_hbm.at[cid, pl.ds(sid * _RZ, _RZ)]
        )

    @pl.when(sid == last_full)
    def _():
        pltpu.sync_copy(
            acc.at[pl.ds(last_full * _RZ, rem)],
            out_hbm.at[cid, pl.ds(last_full * _RZ, rem)],
        )


@functools.cache
def _sc_scatter():
    return pl.kernel(
        _sc_body,
        out_type=jax.ShapeDtypeStruct((_NC, _N, _D), jnp.float32),
        mesh=plsc.VectorSubcoreMesh(
            core_axis_name="c", subcore_axis_name="s", num_cores=_NC, num_subcores=_NS
        ),
        compiler_params=pltpu.CompilerParams(needs_layout_passes=False),
        scratch_types=[
            pltpu.VMEM((_NP, 2, _C), jnp.int32),
            pltpu.VMEM((_NP, _C), jnp.float32),
            pltpu.VMEM((_NR, _C, _D), jnp.float32),
            pltpu.VMEM_SHARED((_NPAD, _D), jnp.float32),
        ] + [pltpu.SemaphoreType.DMA] * (_NP + 2 * _NR),
    )


def _add_body(p_ref, o_ref):
    o_ref[...] = p_ref[0] + p_ref[1]


def _merge_partials(partials):
    ba = 1000
    return pl.pallas_call(
        _add_body,
        grid=(_N // ba,),
        in_specs=[pl.BlockSpec((_NC, ba, _D), lambda i: (0, i, 0))],
        out_specs=pl.BlockSpec((ba, _D), lambda i: (i, 0)),
        out_shape=jax.ShapeDtypeStruct((_N, _D), jnp.float32),
    )(partials)


def kernel(node_features, edge_index, edge_type, edge_norm, weight):
    src = edge_index[0]
    dst = edge_index[1]
    nchunks = _EPAD // _C
    npad = _EPAD - _E
    g = edge_type.astype(jnp.int32) * _N + src.astype(jnp.int32)
    g = jnp.concatenate([g, jnp.zeros((npad,), jnp.int32)])
    d = jnp.concatenate(
        [dst.astype(jnp.int32), jnp.full((npad,), _DUMP, jnp.int32)]
    )
    nrm = jnp.concatenate([edge_norm.astype(jnp.float32), jnp.zeros((npad,), jnp.float32)])
    pack = jnp.stack([g.reshape(nchunks, _C), d.reshape(nchunks, _C)], axis=1)
    normc = nrm.reshape(nchunks, _C)
    y = _relation_matmul(node_features, weight).reshape(_R * _N, _D)
    zeros = jnp.zeros((_RZ, _D), jnp.float32)
    partials = _sc_scatter()(pack, normc, y, zeros)
    return _merge_partials(partials)


# speedup vs baseline: 1.7720x; 1.1797x over previous
import functools

import jax
import jax.numpy as jnp
from jax import lax
from jax.experimental import pallas as pl
from jax.experimental.pallas import tpu as pltpu
from jax.experimental.pallas import tpu_sc as plsc

_N = 10000
_E = 320000
_D = 128
_R = 16

_NC = 2
_NS = 16
_NW = _NC * _NS
_C = 112
_NCHUNK = 90
_EPW = _NCHUNK * _C
_EPAD = _EPW * _NW
_NR = 3
_NP = 6
_NPAD = 10112
_DUMP = _NPAD - 8
_RZ = _NPAD // _NS


def _mm_body(h_ref, w_ref, y_ref):
    y_ref[0] = jnp.dot(h_ref[...], w_ref[0], preferred_element_type=jnp.float32)


def _relation_matmul(node_features, weight):
    bn = 2000
    return pl.pallas_call(
        _mm_body,
        grid=(_N // bn, _R),
        in_specs=[
            pl.BlockSpec((bn, _D), lambda i, r: (i, 0)),
            pl.BlockSpec((1, _D, _D), lambda i, r: (r, 0, 0)),
        ],
        out_specs=pl.BlockSpec((1, bn, _D), lambda i, r: (r, i, 0)),
        out_shape=jax.ShapeDtypeStruct((_R, _N, _D), jnp.float32),
    )(node_features, weight)


def _sc_body(pack_hbm, norm_hbm, y_hbm, zeros_hbm, out_hbm,
             pack_v, norm_v, rows_v, acc, *sems):
    cid = lax.axis_index("c")
    sid = lax.axis_index("s")
    wid = sid * _NC + cid
    base = wid * _NCHUNK
    lsem = sems[0:_NP]
    gsem = sems[_NP:_NP + _NR]
    ssem = sems[_NP + _NR:_NP + 2 * _NR]

    pltpu.sync_copy(zeros_hbm, acc.at[pl.ds(sid * _RZ, _RZ)])
    plsc.subcore_barrier()

    def start_load(p, k):
        pltpu.async_copy(pack_hbm.at[base + k], pack_v.at[p], lsem[p])
        pltpu.async_copy(norm_hbm.at[base + k], norm_v.at[p], lsem[p])

    def wait_load(p):
        pltpu.make_async_copy(pack_hbm.at[0], pack_v.at[p], lsem[p]).wait()
        pltpu.make_async_copy(norm_hbm.at[0], norm_v.at[p], lsem[p]).wait()

    def start_gather(s, p):
        pltpu.async_copy(y_hbm.at[pack_v.at[p, 0]], rows_v.at[s], gsem[s])

    def wait_gather(s):
        pltpu.make_async_copy(y_hbm.at[pl.ds(0, _C)], rows_v.at[s], gsem[s]).wait()

    def start_scatter(s, p):
        pltpu.async_copy(rows_v.at[s], acc.at[pack_v.at[p, 1]], ssem[s], add=True)

    def wait_scatter(s):
        pltpu.make_async_copy(y_hbm.at[pl.ds(0, _C)], rows_v.at[s], ssem[s]).wait()

    def scale(s, p):
        def scale_body(i, carry):
            for u in range(4):
                e = i * 4 + u
                nv = plsc.load_gather(
                    norm_v,
                    [jnp.full((16,), p, jnp.int32), jnp.full((16,), e, jnp.int32)],
                )
                for j in range(_D // 16):
                    sl = pl.ds(j * 16, 16)
                    rows_v[s, e, sl] = rows_v[s, e, sl] * nv
            return carry

        lax.fori_loop(0, _C // 4, scale_body, 0)

    for k in range(4):
        start_load(k, k)
    wait_load(0)
    start_gather(0, 0)

    for k in range(6):
        s, p = k % _NR, k % _NP
        s1, p1 = (k + 1) % _NR, (k + 1) % _NP
        if k >= 2:
            wait_scatter(s1)
        wait_load(p1)
        start_gather(s1, p1)
        start_load((k + 4) % _NP, k + 4)
        wait_gather(s)
        scale(s, p)
        start_scatter(s, p)

    def six_body(q, carry):
        for u in range(6):
            k = q * 6 + u
            s, p = u % _NR, u
            s1, p1 = (u + 1) % _NR, (u + 1) % _NP

            @pl.when(k + 1 < _NCHUNK)
            def _():
                wait_scatter(s1)
                wait_load(p1)
                start_gather(s1, p1)

            @pl.when(k + 4 < _NCHUNK)
            def _():
                start_load((u + 4) % _NP, k + 4)

            wait_gather(s)
            scale(s, p)
            start_scatter(s, p)
        return carry

    lax.fori_loop(1, _NCHUNK // 6, six_body, 0)

    for s in range(_NR):
        wait_scatter(s)

    plsc.subcore_barrier()

    last_full = _N // _RZ
    rem = _N - last_full * _RZ

    @pl.when(sid < last_full)
    def _():
        pltpu.sync_copy(
            acc.at[pl.ds(sid * _RZ, _RZ)], out_hbm.at[cid, pl.ds(sid * _RZ, _RZ)]
        )

    @pl.when(sid == last_full)
    def _():
        pltpu.sync_copy(
            acc.at[pl.ds(last_full * _RZ, rem)],
            out_hbm.at[cid, pl.ds(last_full * _RZ, rem)],
        )


@functools.cache
def _sc_scatter():
    return pl.kernel(
        _sc_body,
        out_type=jax.ShapeDtypeStruct((_NC, _N, _D), jnp.float32),
        mesh=plsc.VectorSubcoreMesh(
            core_axis_name="c", subcore_axis_name="s", num_cores=_NC, num_subcores=_NS
        ),
        compiler_params=pltpu.CompilerParams(needs_layout_passes=False),
        scratch_types=[
            pltpu.VMEM((_NP, 2, _C), jnp.int32),
            pltpu.VMEM((_NP, _C), jnp.float32),
            pltpu.VMEM((_NR, _C, _D), jnp.float32),
            pltpu.VMEM_SHARED((_NPAD, _D), jnp.float32),
        ] + [pltpu.SemaphoreType.DMA] * (_NP + 2 * _NR),
    )


def _add_body(p_ref, o_ref):
    o_ref[...] = p_ref[0] + p_ref[1]


def _merge_partials(partials):
    ba = 1000
    return pl.pallas_call(
        _add_body,
        grid=(_N // ba,),
        in_specs=[pl.BlockSpec((_NC, ba, _D), lambda i: (0, i, 0))],
        out_specs=pl.BlockSpec((ba, _D), lambda i: (i, 0)),
        out_shape=jax.ShapeDtypeStruct((_N, _D), jnp.float32),
    )(partials)


def kernel(node_features, edge_index, edge_type, edge_norm, weight):
    src = edge_index[0]
    dst = edge_index[1]
    nchunks = _EPAD // _C
    npad = _EPAD - _E
    g = edge_type.astype(jnp.int32) * _N + src.astype(jnp.int32)
    g = jnp.concatenate([g, jnp.zeros((npad,), jnp.int32)])
    d = jnp.concatenate(
        [dst.astype(jnp.int32), jnp.full((npad,), _DUMP, jnp.int32)]
    )
    nrm = jnp.concatenate([edge_norm.astype(jnp.float32), jnp.zeros((npad,), jnp.float32)])
    pack = jnp.stack([g.reshape(nchunks, _C), d.reshape(nchunks, _C)], axis=1)
    normc = nrm.reshape(nchunks, _C)
    y = _relation_matmul(node_features, weight).reshape(_R * _N, _D)
    zeros = jnp.zeros((_RZ, _D), jnp.float32)
    partials = _sc_scatter()(pack, normc, y, zeros)
    return _merge_partials(partials)


# speedup vs baseline: 1.9746x; 1.1143x over previous
import functools

import jax
import jax.numpy as jnp
from jax import lax
from jax.experimental import pallas as pl
from jax.experimental.pallas import tpu as pltpu
from jax.experimental.pallas import tpu_sc as plsc

_N = 10000
_E = 320000
_D = 128
_R = 16

_NC = 2
_NS = 16
_NW = _NC * _NS
_C = 112
_NCHUNK = 90
_K0 = 120
_K1 = 2 * _NCHUNK - _K0
_EPW = _NCHUNK * _C
_EPAD = 2 * _NCHUNK * _C * _NS
_NR = 3
_NP = 6
_NPAD = 10112
_DUMP = _NPAD - 8
_RZ = _NPAD // _NS


def _mm_body(h_ref, w_ref, y_ref):
    y_ref[0] = jnp.dot(h_ref[...], w_ref[0], preferred_element_type=jnp.float32)


def _relation_matmul(node_features, weight):
    bn = 2000
    return pl.pallas_call(
        _mm_body,
        grid=(_N // bn, _R),
        in_specs=[
            pl.BlockSpec((bn, _D), lambda i, r: (i, 0)),
            pl.BlockSpec((1, _D, _D), lambda i, r: (r, 0, 0)),
        ],
        out_specs=pl.BlockSpec((1, bn, _D), lambda i, r: (r, i, 0)),
        out_shape=jax.ShapeDtypeStruct((_R, _N, _D), jnp.float32),
    )(node_features, weight)


def _sc_body(pack_hbm, norm_hbm, y_hbm, zeros_hbm, out_hbm,
             pack_v, norm_v, rows_v, acc, *sems):
    cid = lax.axis_index("c")
    sid = lax.axis_index("s")
    base = sid * (2 * _NCHUNK) + cid * _K0
    nself = lax.select(cid == 0, jnp.int32(_K0), jnp.int32(_K1))
    lsem = sems[0:_NP]
    gsem = sems[_NP:_NP + _NR]
    ssem = sems[_NP + _NR:_NP + 2 * _NR]

    pltpu.sync_copy(zeros_hbm, acc.at[pl.ds(sid * _RZ, _RZ)])
    plsc.subcore_barrier()

    def start_load(p, k):
        pltpu.async_copy(pack_hbm.at[base + k], pack_v.at[p], lsem[p])
        pltpu.async_copy(norm_hbm.at[base + k], norm_v.at[p], lsem[p])

    def wait_load(p):
        pltpu.make_async_copy(pack_hbm.at[0], pack_v.at[p], lsem[p]).wait()
        pltpu.make_async_copy(norm_hbm.at[0], norm_v.at[p], lsem[p]).wait()

    def start_gather(s, p):
        pltpu.async_copy(y_hbm.at[pack_v.at[p, 0]], rows_v.at[s], gsem[s])

    def wait_gather(s):
        pltpu.make_async_copy(y_hbm.at[pl.ds(0, _C)], rows_v.at[s], gsem[s]).wait()

    def start_scatter(s, p):
        pltpu.async_copy(rows_v.at[s], acc.at[pack_v.at[p, 1]], ssem[s], add=True)

    def wait_scatter(s):
        pltpu.make_async_copy(y_hbm.at[pl.ds(0, _C)], rows_v.at[s], ssem[s]).wait()

    def scale(s, p):
        def scale_body(i, carry):
            for u in range(4):
                e = i * 4 + u
                nv = plsc.load_gather(
                    norm_v,
                    [jnp.full((16,), p, jnp.int32), jnp.full((16,), e, jnp.int32)],
                )
                for j in range(_D // 16):
                    sl = pl.ds(j * 16, 16)
                    rows_v[s, e, sl] = rows_v[s, e, sl] * nv
            return carry

        lax.fori_loop(0, _C // 4, scale_body, 0)

    for k in range(4):
        start_load(k, k)
    wait_load(0)
    start_gather(0, 0)

    for k in range(6):
        s, p = k % _NR, k % _NP
        s1, p1 = (k + 1) % _NR, (k + 1) % _NP
        if k >= 2:
            wait_scatter(s1)
        wait_load(p1)
        start_gather(s1, p1)
        start_load((k + 4) % _NP, k + 4)
        wait_gather(s)
        scale(s, p)
        start_scatter(s, p)

    def six_body(q, carry):
        for u in range(6):
            k = q * 6 + u
            s, p = u % _NR, u
            s1, p1 = (u + 1) % _NR, (u + 1) % _NP

            @pl.when(k + 1 < nself)
            def _():
                wait_scatter(s1)
                wait_load(p1)
                start_gather(s1, p1)

            @pl.when(k + 4 < nself)
            def _():
                start_load((u + 4) % _NP, k + 4)

            wait_gather(s)
            scale(s, p)
            start_scatter(s, p)
        return carry

    lax.fori_loop(1, nself // 6, six_body, 0)

    for s in range(_NR):
        wait_scatter(s)

    plsc.subcore_barrier()

    last_full = _N // _RZ
    rem = _N - last_full * _RZ

    @pl.when(sid < last_full)
    def _():
        pltpu.sync_copy(
            acc.at[pl.ds(sid * _RZ, _RZ)], out_hbm.at[cid, pl.ds(sid * _RZ, _RZ)]
        )

    @pl.when(sid == last_full)
    def _():
        pltpu.sync_copy(
            acc.at[pl.ds(last_full * _RZ, rem)],
            out_hbm.at[cid, pl.ds(last_full * _RZ, rem)],
        )


@functools.cache
def _sc_scatter():
    return pl.kernel(
        _sc_body,
        out_type=jax.ShapeDtypeStruct((_NC, _N, _D), jnp.float32),
        mesh=plsc.VectorSubcoreMesh(
            core_axis_name="c", subcore_axis_name="s", num_cores=_NC, num_subcores=_NS
        ),
        compiler_params=pltpu.CompilerParams(needs_layout_passes=False),
        scratch_types=[
            pltpu.VMEM((_NP, 2, _C), jnp.int32),
            pltpu.VMEM((_NP, _C), jnp.float32),
            pltpu.VMEM((_NR, _C, _D), jnp.float32),
            pltpu.VMEM_SHARED((_NPAD, _D), jnp.float32),
        ] + [pltpu.SemaphoreType.DMA] * (_NP + 2 * _NR),
    )


def _add_body(p_ref, o_ref):
    o_ref[...] = p_ref[0] + p_ref[1]


def _merge_partials(partials):
    ba = 1000
    return pl.pallas_call(
        _add_body,
        grid=(_N // ba,),
        in_specs=[pl.BlockSpec((_NC, ba, _D), lambda i: (0, i, 0))],
        out_specs=pl.BlockSpec((ba, _D), lambda i: (i, 0)),
        out_shape=jax.ShapeDtypeStruct((_N, _D), jnp.float32),
    )(partials)


def kernel(node_features, edge_index, edge_type, edge_norm, weight):
    src = edge_index[0]
    dst = edge_index[1]
    nchunks = _EPAD // _C
    npad = _EPAD - _E
    g = edge_type.astype(jnp.int32) * _N + src.astype(jnp.int32)
    g = jnp.concatenate([g, jnp.zeros((npad,), jnp.int32)])
    d = jnp.concatenate(
        [dst.astype(jnp.int32), jnp.full((npad,), _DUMP, jnp.int32)]
    )
    nrm = jnp.concatenate([edge_norm.astype(jnp.float32), jnp.zeros((npad,), jnp.float32)])
    pack = jnp.stack([g.reshape(nchunks, _C), d.reshape(nchunks, _C)], axis=1)
    normc = nrm.reshape(nchunks, _C)
    y = _relation_matmul(node_features, weight).reshape(_R * _N, _D)
    zeros = jnp.zeros((_RZ, _D), jnp.float32)
    partials = _sc_scatter()(pack, normc, y, zeros)
    return _merge_partials(partials)
